# baseline (device time: 42350 ns/iter reference)
import jax
import jax.numpy as jnp
from jax import lax
from jax.experimental import pallas as pl
from jax.experimental.pallas import tpu as pltpu

N_DEV = 32
B, SQ, DMODEL = 2, 256, 512
HQ_PER, DH = 4, 64
COLS = HQ_PER * DH
TOK = B * SQ
ROWS = TOK // N_DEV
BLK = 64


def kernel(x, Wq, K_ext, V_ext, Wo):
    i = lax.axis_index("i")
    Wq_i = lax.dynamic_slice(Wq, (0, i * COLS), (DMODEL, COLS))
    Wo_i = lax.dynamic_slice(Wo, (i * COLS, 0), (COLS, DMODEL))
    x2 = x.reshape(TOK, DMODEL)
    k2 = K_ext.reshape(TOK, HQ_PER, DH)
    v2 = V_ext.reshape(TOK, HQ_PER, DH)

    def body(x_ref, wq_ref, k_ref, v_ref, wo_ref, out_ref,
             part_ref, ctx_ref, gather_ref, red_ref,
             p1_send, p1_recv, p2_send, p2_recv):
        my = lax.axis_index("i")

        barrier = pltpu.get_barrier_semaphore()
        for j in range(1, N_DEV):
            pl.semaphore_signal(
                barrier, inc=1,
                device_id=(lax.rem(my + j, N_DEV),),
                device_id_type=pl.DeviceIdType.MESH,
            )
        pl.semaphore_wait(barrier, N_DEV - 1)

        xb = x_ref[...].astype(jnp.bfloat16)
        wqb = wq_ref[...].astype(jnp.bfloat16)
        q = jnp.dot(xb, wqb, preferred_element_type=jnp.float32) * 0.125
        row = lax.broadcasted_iota(jnp.int32, (TOK, TOK), 0) // BLK
        col = lax.broadcasted_iota(jnp.int32, (TOK, TOK), 1) // BLK
        mask = row == col
        for h in range(HQ_PER):
            qh = q[:, h * DH:(h + 1) * DH].astype(jnp.bfloat16)
            kh = k_ref[:, h, :].astype(jnp.bfloat16)
            vh = v_ref[:, h, :].astype(jnp.bfloat16)
            s = lax.dot_general(
                qh, kh, (((1,), (1,)), ((), ())),
                preferred_element_type=jnp.float32,
            )
            s = jnp.where(mask, s, -1e9)
            m = jnp.max(s, axis=1, keepdims=True)
            w = jnp.exp(s - m)
            w = w / jnp.sum(w, axis=1, keepdims=True)
            ctx_ref[:, h * DH:(h + 1) * DH] = jnp.dot(
                w.astype(jnp.bfloat16), vh,
                preferred_element_type=jnp.float32,
            ).astype(jnp.bfloat16)
        part_ref[...] = jnp.dot(
            ctx_ref[...], wo_ref[...].astype(jnp.bfloat16),
            preferred_element_type=jnp.float32,
        )

        p1 = []
        for j in range(1, N_DEV):
            tgt = lax.rem(my + j, N_DEV)
            rd = pltpu.make_async_remote_copy(
                src_ref=part_ref.at[pl.ds(tgt * ROWS, ROWS), :],
                dst_ref=gather_ref.at[j - 1],
                send_sem=p1_send.at[j - 1],
                recv_sem=p1_recv.at[j - 1],
                device_id=(tgt,),
                device_id_type=pl.DeviceIdType.MESH,
            )
            rd.start()
            p1.append(rd)
        for o in range(1, N_DEV):
            org = lax.rem(my + N_DEV - o, N_DEV)
            rd = pltpu.make_async_remote_copy(
                src_ref=part_ref.at[pl.ds(0, ROWS), :],
                dst_ref=gather_ref.at[o - 1],
                send_sem=p1_send.at[o - 1],
                recv_sem=p1_recv.at[o - 1],
                device_id=(org,),
                device_id_type=pl.DeviceIdType.MESH,
            )
            rd.wait_recv()

        red = part_ref[pl.ds(my * ROWS, ROWS), :] + jnp.sum(
            gather_ref[...], axis=0
        )
        red_ref[...] = red
        out_ref[pl.ds(my * ROWS, ROWS), :] = red

        p2 = []
        for j in range(1, N_DEV):
            tgt = lax.rem(my + j, N_DEV)
            rd = pltpu.make_async_remote_copy(
                src_ref=red_ref,
                dst_ref=out_ref.at[pl.ds(my * ROWS, ROWS), :],
                send_sem=p2_send.at[j - 1],
                recv_sem=p2_recv.at[j - 1],
                device_id=(tgt,),
                device_id_type=pl.DeviceIdType.MESH,
            )
            rd.start()
            p2.append(rd)
        for o in range(1, N_DEV):
            org = lax.rem(my + N_DEV - o, N_DEV)
            rd = pltpu.make_async_remote_copy(
                src_ref=red_ref,
                dst_ref=out_ref.at[pl.ds(org * ROWS, ROWS), :],
                send_sem=p2_send.at[o - 1],
                recv_sem=p2_recv.at[o - 1],
                device_id=(org,),
                device_id_type=pl.DeviceIdType.MESH,
            )
            rd.wait_recv()
        for rd in p1:
            rd.wait_send()
        for rd in p2:
            rd.wait_send()

    out2 = pl.pallas_call(
        body,
        out_shape=jax.ShapeDtypeStruct((TOK, DMODEL), jnp.float32),
        in_specs=[pl.BlockSpec(memory_space=pltpu.VMEM)] * 5,
        out_specs=pl.BlockSpec(memory_space=pltpu.VMEM),
        scratch_shapes=[
            pltpu.VMEM((TOK, DMODEL), jnp.float32),
            pltpu.VMEM((TOK, COLS), jnp.bfloat16),
            pltpu.VMEM((N_DEV - 1, ROWS, DMODEL), jnp.float32),
            pltpu.VMEM((ROWS, DMODEL), jnp.float32),
            pltpu.SemaphoreType.DMA((N_DEV - 1,)),
            pltpu.SemaphoreType.DMA((N_DEV - 1,)),
            pltpu.SemaphoreType.DMA((N_DEV - 1,)),
            pltpu.SemaphoreType.DMA((N_DEV - 1,)),
        ],
        compiler_params=pltpu.CompilerParams(collective_id=0),
    )(x2, Wq_i, k2, v2, Wo_i)
    return out2.reshape(B, SQ, DMODEL)


# device time: 42323 ns/iter; 1.0006x vs baseline; 1.0006x over previous
import jax
import jax.numpy as jnp
from jax import lax
from jax.experimental import pallas as pl
from jax.experimental.pallas import tpu as pltpu

N_DEV = 32
B, SQ, DMODEL = 2, 256, 512
HQ_PER, DH = 4, 64
COLS = HQ_PER * DH
TOK = B * SQ
ROWS = TOK // N_DEV
BLK = 64


def kernel(x, Wq, K_ext, V_ext, Wo):
    i = lax.axis_index("i")
    Wq_i = lax.dynamic_slice(Wq, (0, i * COLS), (DMODEL, COLS))
    Wo_i = lax.dynamic_slice(Wo, (i * COLS, 0), (COLS, DMODEL))
    x2 = x.reshape(TOK, DMODEL)
    k2 = K_ext.reshape(TOK, HQ_PER, DH)
    v2 = V_ext.reshape(TOK, HQ_PER, DH)

    def body(x_ref, wq_ref, k_ref, v_ref, wo_ref, out_ref,
             part_ref, ctx_ref, gather_ref, red_ref,
             p1_send, p1_recv, p2_send, p2_recv):
        my = lax.axis_index("i")

        barrier = pltpu.get_barrier_semaphore()
        for j in range(1, N_DEV):
            pl.semaphore_signal(
                barrier, inc=1,
                device_id=(lax.rem(my + j, N_DEV),),
                device_id_type=pl.DeviceIdType.MESH,
            )

        xb = x_ref[...].astype(jnp.bfloat16)
        wqb = wq_ref[...].astype(jnp.bfloat16)
        q = jnp.dot(xb, wqb, preferred_element_type=jnp.float32) * 0.125
        row = lax.broadcasted_iota(jnp.int32, (TOK, TOK), 0) // BLK
        col = lax.broadcasted_iota(jnp.int32, (TOK, TOK), 1) // BLK
        mask = row == col
        for h in range(HQ_PER):
            qh = q[:, h * DH:(h + 1) * DH].astype(jnp.bfloat16)
            kh = k_ref[:, h, :].astype(jnp.bfloat16)
            vh = v_ref[:, h, :].astype(jnp.bfloat16)
            s = lax.dot_general(
                qh, kh, (((1,), (1,)), ((), ())),
                preferred_element_type=jnp.float32,
            )
            s = jnp.where(mask, s, -1e9)
            m = jnp.max(s, axis=1, keepdims=True)
            w = jnp.exp(s - m)
            w = w / jnp.sum(w, axis=1, keepdims=True)
            ctx_ref[:, h * DH:(h + 1) * DH] = jnp.dot(
                w.astype(jnp.bfloat16), vh,
                preferred_element_type=jnp.float32,
            ).astype(jnp.bfloat16)
        part_ref[...] = jnp.dot(
            ctx_ref[...], wo_ref[...].astype(jnp.bfloat16),
            preferred_element_type=jnp.float32,
        )

        pl.semaphore_wait(barrier, N_DEV - 1)

        p1 = []
        for j in range(1, N_DEV):
            tgt = lax.rem(my + j, N_DEV)
            rd = pltpu.make_async_remote_copy(
                src_ref=part_ref.at[pl.ds(tgt * ROWS, ROWS), :],
                dst_ref=gather_ref.at[j - 1],
                send_sem=p1_send.at[j - 1],
                recv_sem=p1_recv.at[j - 1],
                device_id=(tgt,),
                device_id_type=pl.DeviceIdType.MESH,
            )
            rd.start()
            p1.append(rd)
        for o in range(1, N_DEV):
            org = lax.rem(my + N_DEV - o, N_DEV)
            rd = pltpu.make_async_remote_copy(
                src_ref=part_ref.at[pl.ds(0, ROWS), :],
                dst_ref=gather_ref.at[o - 1],
                send_sem=p1_send.at[o - 1],
                recv_sem=p1_recv.at[o - 1],
                device_id=(org,),
                device_id_type=pl.DeviceIdType.MESH,
            )
            rd.wait_recv()

        red = part_ref[pl.ds(my * ROWS, ROWS), :] + jnp.sum(
            gather_ref[...], axis=0
        )
        red_ref[...] = red
        out_ref[pl.ds(my * ROWS, ROWS), :] = red

        p2 = []
        for j in range(1, N_DEV):
            tgt = lax.rem(my + j, N_DEV)
            rd = pltpu.make_async_remote_copy(
                src_ref=red_ref,
                dst_ref=out_ref.at[pl.ds(my * ROWS, ROWS), :],
                send_sem=p2_send.at[j - 1],
                recv_sem=p2_recv.at[j - 1],
                device_id=(tgt,),
                device_id_type=pl.DeviceIdType.MESH,
            )
            rd.start()
            p2.append(rd)
        for o in range(1, N_DEV):
            org = lax.rem(my + N_DEV - o, N_DEV)
            rd = pltpu.make_async_remote_copy(
                src_ref=red_ref,
                dst_ref=out_ref.at[pl.ds(org * ROWS, ROWS), :],
                send_sem=p2_send.at[o - 1],
                recv_sem=p2_recv.at[o - 1],
                device_id=(org,),
                device_id_type=pl.DeviceIdType.MESH,
            )
            rd.wait_recv()
        for rd in p1:
            rd.wait_send()
        for rd in p2:
            rd.wait_send()

    out2 = pl.pallas_call(
        body,
        out_shape=jax.ShapeDtypeStruct((TOK, DMODEL), jnp.float32),
        in_specs=[pl.BlockSpec(memory_space=pltpu.VMEM)] * 5,
        out_specs=pl.BlockSpec(memory_space=pltpu.VMEM),
        scratch_shapes=[
            pltpu.VMEM((TOK, DMODEL), jnp.float32),
            pltpu.VMEM((TOK, COLS), jnp.bfloat16),
            pltpu.VMEM((N_DEV - 1, ROWS, DMODEL), jnp.float32),
            pltpu.VMEM((ROWS, DMODEL), jnp.float32),
            pltpu.SemaphoreType.DMA((N_DEV - 1,)),
            pltpu.SemaphoreType.DMA((N_DEV - 1,)),
            pltpu.SemaphoreType.DMA((N_DEV - 1,)),
            pltpu.SemaphoreType.DMA((N_DEV - 1,)),
        ],
        compiler_params=pltpu.CompilerParams(collective_id=0),
    )(x2, Wq_i, k2, v2, Wo_i)
    return out2.reshape(B, SQ, DMODEL)


# device time: 32153 ns/iter; 1.3171x vs baseline; 1.3163x over previous
import jax
import jax.numpy as jnp
from jax import lax
from jax.experimental import pallas as pl
from jax.experimental.pallas import tpu as pltpu

N_DEV = 32
B, SQ, DMODEL = 2, 256, 512
HQ_PER, DH = 4, 64
COLS = HQ_PER * DH
TOK = B * SQ
ROWS = TOK // N_DEV
BLK = 64


def kernel(x, Wq, K_ext, V_ext, Wo):
    i = lax.axis_index("i")
    Wq_i = lax.dynamic_slice(Wq, (0, i * COLS), (DMODEL, COLS))
    Wo_i = lax.dynamic_slice(Wo, (i * COLS, 0), (COLS, DMODEL))
    x2 = x.reshape(TOK, DMODEL)
    k2 = K_ext.reshape(TOK, HQ_PER, DH)
    v2 = V_ext.reshape(TOK, HQ_PER, DH)

    def body(x_ref, wq_ref, k_ref, v_ref, wo_ref, out_ref,
             part_ref, ctx_ref, send_ref, gather_ref, red_ref, rgather_ref,
             p1_send, p1_recv, p2_send, p2_recv):
        my = lax.axis_index("i")

        barrier = pltpu.get_barrier_semaphore()
        for j in range(1, N_DEV):
            pl.semaphore_signal(
                barrier, inc=1,
                device_id=(lax.rem(my + j, N_DEV),),
                device_id_type=pl.DeviceIdType.MESH,
            )
        pl.semaphore_wait(barrier, N_DEV - 1)

        xb = x_ref[...].astype(jnp.bfloat16)
        wqb = wq_ref[...].astype(jnp.bfloat16)
        q = jnp.dot(xb, wqb, preferred_element_type=jnp.float32) * 0.125
        row = lax.broadcasted_iota(jnp.int32, (TOK, TOK), 0) // BLK
        col = lax.broadcasted_iota(jnp.int32, (TOK, TOK), 1) // BLK
        mask = row == col
        for h in range(HQ_PER):
            qh = q[:, h * DH:(h + 1) * DH].astype(jnp.bfloat16)
            kh = k_ref[:, h, :].astype(jnp.bfloat16)
            vh = v_ref[:, h, :].astype(jnp.bfloat16)
            s = lax.dot_general(
                qh, kh, (((1,), (1,)), ((), ())),
                preferred_element_type=jnp.float32,
            )
            s = jnp.where(mask, s, -1e9)
            m = jnp.max(s, axis=1, keepdims=True)
            w = jnp.exp(s - m)
            w = w / jnp.sum(w, axis=1, keepdims=True)
            ctx_ref[:, h * DH:(h + 1) * DH] = jnp.dot(
                w.astype(jnp.bfloat16), vh,
                preferred_element_type=jnp.float32,
            ).astype(jnp.bfloat16)
        part_ref[...] = jnp.dot(
            ctx_ref[...], wo_ref[...].astype(jnp.bfloat16),
            preferred_element_type=jnp.float32,
        ).astype(jnp.bfloat16)

        for j in range(1, N_DEV):
            tgt = lax.rem(my + j, N_DEV)
            send_ref[j - 1] = part_ref[pl.ds(tgt * ROWS, ROWS), :]

        p1 = []
        for j in range(1, N_DEV):
            tgt = lax.rem(my + j, N_DEV)
            rd = pltpu.make_async_remote_copy(
                src_ref=send_ref.at[j - 1],
                dst_ref=gather_ref.at[j - 1],
                send_sem=p1_send.at[j - 1],
                recv_sem=p1_recv.at[j - 1],
                device_id=(tgt,),
                device_id_type=pl.DeviceIdType.MESH,
            )
            rd.start()
            p1.append(rd)
        for o in range(1, N_DEV):
            org = lax.rem(my + N_DEV - o, N_DEV)
            rd = pltpu.make_async_remote_copy(
                src_ref=send_ref.at[o - 1],
                dst_ref=gather_ref.at[o - 1],
                send_sem=p1_send.at[o - 1],
                recv_sem=p1_recv.at[o - 1],
                device_id=(org,),
                device_id_type=pl.DeviceIdType.MESH,
            )
            rd.wait_recv()

        red = (
            part_ref[pl.ds(my * ROWS, ROWS), :].astype(jnp.float32)
            + jnp.sum(gather_ref[...].astype(jnp.float32), axis=0)
        ).astype(jnp.bfloat16)
        red_ref[...] = red
        out_ref[pl.ds(my * ROWS, ROWS), :] = red

        p2 = []
        for j in range(1, N_DEV):
            tgt = lax.rem(my + j, N_DEV)
            rd = pltpu.make_async_remote_copy(
                src_ref=red_ref,
                dst_ref=rgather_ref.at[j - 1],
                send_sem=p2_send.at[j - 1],
                recv_sem=p2_recv.at[j - 1],
                device_id=(tgt,),
                device_id_type=pl.DeviceIdType.MESH,
            )
            rd.start()
            p2.append(rd)
        for o in range(1, N_DEV):
            org = lax.rem(my + N_DEV - o, N_DEV)
            rd = pltpu.make_async_remote_copy(
                src_ref=red_ref,
                dst_ref=rgather_ref.at[o - 1],
                send_sem=p2_send.at[o - 1],
                recv_sem=p2_recv.at[o - 1],
                device_id=(org,),
                device_id_type=pl.DeviceIdType.MESH,
            )
            rd.wait_recv()
            out_ref[pl.ds(org * ROWS, ROWS), :] = rgather_ref[o - 1]
        for rd in p1:
            rd.wait_send()
        for rd in p2:
            rd.wait_send()

    out2 = pl.pallas_call(
        body,
        out_shape=jax.ShapeDtypeStruct((TOK, DMODEL), jnp.bfloat16),
        in_specs=[pl.BlockSpec(memory_space=pltpu.VMEM)] * 5,
        out_specs=pl.BlockSpec(memory_space=pltpu.VMEM),
        scratch_shapes=[
            pltpu.VMEM((TOK, DMODEL), jnp.bfloat16),
            pltpu.VMEM((TOK, COLS), jnp.bfloat16),
            pltpu.VMEM((N_DEV - 1, ROWS, DMODEL), jnp.bfloat16),
            pltpu.VMEM((N_DEV - 1, ROWS, DMODEL), jnp.bfloat16),
            pltpu.VMEM((ROWS, DMODEL), jnp.bfloat16),
            pltpu.VMEM((N_DEV - 1, ROWS, DMODEL), jnp.bfloat16),
            pltpu.SemaphoreType.DMA((N_DEV - 1,)),
            pltpu.SemaphoreType.DMA((N_DEV - 1,)),
            pltpu.SemaphoreType.DMA((N_DEV - 1,)),
            pltpu.SemaphoreType.DMA((N_DEV - 1,)),
        ],
        compiler_params=pltpu.CompilerParams(collective_id=0),
    )(x2, Wq_i, k2, v2, Wo_i)
    return out2.reshape(B, SQ, DMODEL)
